# baseline (speedup 1.0000x reference)
import functools

import jax
import jax.numpy as jnp
from jax import lax
from jax.experimental import pallas as pl
from jax.experimental.pallas import tpu as pltpu
from jax.experimental.pallas import tpu_sc as plsc

B = 16384
F = 26
V = 100000
D = 32

NC, NS, L = 2, 16, 16
NW = NC * NS
ROWS_W = (B * F) // NW
CH = 128
NCH = ROWS_W // CH
NBUF = 8
K = 4


def _kernel_body(idx_hbm, tab_hbm, out_hbm, idx_v, *scratch):
    bufs = scratch[:NBUF]
    gsems = scratch[NBUF:2 * NBUF]
    wsems = scratch[2 * NBUF:3 * NBUF]
    wid = lax.axis_index("s") * NC + lax.axis_index("c")
    base_row = wid * ROWS_W

    pltpu.sync_copy(idx_hbm.at[wid], idx_v)

    lanes = lax.broadcasted_iota(jnp.int32, (L,), 0)

    def compute_idx(c):
        for s in range(CH // L):
            pos = lanes + (c * CH + s * L)
            off = lax.rem(pos, F) * V
            idx_v[c, pl.ds(s * L, L)] = idx_v[c, pl.ds(s * L, L)] + off

    def start_gather(c, b):
        pltpu.async_copy(tab_hbm.at[idx_v.at[c]], bufs[b], gsems[b])

    for c in range(K):
        compute_idx(c)
        start_gather(c, c % NBUF)

    def ring(g, _):
        for j in range(NBUF):
            i = g + j
            bg = (j + K) % NBUF

            @pl.when(i + K < NCH)
            def _():
                @pl.when(i + K >= NBUF)
                def _():
                    pltpu.make_async_copy(bufs[bg], out_hbm.at[pl.ds(0, CH)], wsems[bg]).wait()
                compute_idx(i + K)
                start_gather(i + K, bg)

            pltpu.make_async_copy(tab_hbm.at[idx_v.at[i]], bufs[j], gsems[j]).wait()
            pltpu.async_copy(bufs[j], out_hbm.at[pl.ds(base_row + i * CH, CH)], wsems[j])
        return 0

    lax.fori_loop(0, NCH // NBUF, lambda g, x: ring(g * NBUF, x), 0, unroll=False)

    for j in range(NBUF):
        pltpu.make_async_copy(bufs[j], out_hbm.at[pl.ds(0, CH)], wsems[j]).wait()


@jax.jit
def _embed(idx, tab):
    mesh = plsc.VectorSubcoreMesh(
        core_axis_name="c", subcore_axis_name="s", num_cores=NC, num_subcores=NS
    )
    scratch = (
        [pltpu.VMEM((NCH, CH), jnp.int32)]
        + [pltpu.VMEM((CH, D), jnp.float32) for _ in range(NBUF)]
        + [pltpu.SemaphoreType.DMA for _ in range(2 * NBUF)]
    )
    return pl.kernel(
        _kernel_body,
        out_type=jax.ShapeDtypeStruct((B * F, D), jnp.float32),
        mesh=mesh,
        scratch_types=scratch,
        compiler_params=pltpu.CompilerParams(use_tc_tiling_on_sc=False),
    )(idx, tab)


def kernel(input, tables):
    idx = input.astype(jnp.int32).reshape(NW, NCH, CH)
    tab = tables.reshape(F * V, D)
    return _embed(idx, tab).reshape(B, F, D)


# speedup vs baseline: 3.6003x; 3.6003x over previous
import jax
import jax.numpy as jnp
from jax import lax
from jax.experimental import pallas as pl
from jax.experimental.pallas import tpu as pltpu
from jax.experimental.pallas import tpu_sc as plsc

B = 16384
F = 26
V = 100000
D = 32

NC, NS, L = 2, 16, 16
NW = NC * NS


CHB = 4096
NCHB = B // CHB


def _kernel_body(idx_hbm, tab_hbm, out_hbm, row_v,
                 idx0, idx1, out0, out1, rsem, isem0, isem1, osem0, osem1):
    d = lax.axis_index("s") * NC + lax.axis_index("c")
    idx_v = (idx0, idx1)
    out_v = (out0, out1)
    isem = (isem0, isem1)
    osem = (osem0, osem1)

    def field(f, _):
        pltpu.async_copy(tab_hbm.at[f, d], row_v, rsem)
        pltpu.async_copy(idx_hbm.at[f, pl.ds(0, CHB)], idx_v[0], isem[0])
        pltpu.make_async_copy(tab_hbm.at[f, d], row_v, rsem).wait()
        for h in range(NCHB):
            p = h % 2
            if h + 1 < NCHB:
                pltpu.async_copy(
                    idx_hbm.at[f, pl.ds((h + 1) * CHB, CHB)], idx_v[1 - p], isem[1 - p]
                )
            pltpu.make_async_copy(
                idx_hbm.at[f, pl.ds(0, CHB)], idx_v[p], isem[p]
            ).wait()
            if h >= 2:
                pltpu.make_async_copy(out_v[p], out_hbm.at[0, 0, pl.ds(0, CHB)],
                                      osem[p]).wait()
            else:
                @pl.when(f > 0)
                def _():
                    pltpu.make_async_copy(out_v[p], out_hbm.at[0, 0, pl.ds(0, CHB)],
                                          osem[p]).wait()

            def chunk(i, _):
                ids = idx_v[p][pl.ds(i * L, L)]
                out_v[p][pl.ds(i * L, L)] = plsc.load_gather(row_v, [ids])
                return 0

            lax.fori_loop(0, CHB // L, chunk, 0, unroll=8)
            pltpu.async_copy(out_v[p], out_hbm.at[f, d, pl.ds(h * CHB, CHB)], osem[p])
        return 0

    lax.fori_loop(0, F, field, 0, unroll=False)
    for p in range(2):
        pltpu.make_async_copy(out_v[p], out_hbm.at[0, 0, pl.ds(0, CHB)], osem[p]).wait()


@jax.jit
def _embed(idx_t, tab_t):
    mesh = plsc.VectorSubcoreMesh(
        core_axis_name="c", subcore_axis_name="s", num_cores=NC, num_subcores=NS
    )
    scratch = (
        [pltpu.VMEM((V,), jnp.float32)]
        + [pltpu.VMEM((CHB,), jnp.int32) for _ in range(2)]
        + [pltpu.VMEM((CHB,), jnp.float32) for _ in range(2)]
        + [pltpu.SemaphoreType.DMA for _ in range(5)]
    )
    return pl.kernel(
        _kernel_body,
        out_type=jax.ShapeDtypeStruct((F, D, B), jnp.float32),
        mesh=mesh,
        scratch_types=scratch,
        compiler_params=pltpu.CompilerParams(
            use_tc_tiling_on_sc=True, needs_layout_passes=False
        ),
    )(idx_t, tab_t)


def kernel(input, tables):
    idx_t = input.astype(jnp.int32).T
    tab_t = jnp.transpose(tables, (0, 2, 1))
    out_t = _embed(idx_t, tab_t)
    return jnp.transpose(out_t, (2, 0, 1))
